# baseline (speedup 1.0000x reference)
import functools

import jax
import jax.numpy as jnp
from jax import lax
from jax.experimental import pallas as pl
from jax.experimental.pallas import tpu as pltpu
from jax.experimental.pallas import tpu_sc as plsc

_N = 10000
_E = 320000
_R = 4
_NUM_NEIGHBORS = 32

_NC = 2
_NS = 16
_NW = _NC * _NS

_G = 128
_E_PAD = 327680
_GW = 80
_PPG = 40
_N_ACC = 10240
_SLAB = _N_ACC // _NS


def _sc_aggregate(d: int):
    mesh = plsc.VectorSubcoreMesh(core_axis_name="c", subcore_axis_name="s")

    @functools.partial(
        pl.kernel,
        out_type=jax.ShapeDtypeStruct((_NC, _N_ACC, d), jnp.float32),
        mesh=mesh,
        scratch_types=[
            pltpu.VMEM((_PPG, _G), jnp.int32),
            pltpu.VMEM((_PPG, _G), jnp.int32),
            pltpu.VMEM((_PPG, _G), jnp.int32),
            pltpu.VMEM((_G, d), jnp.float32),
            pltpu.VMEM((_G, d), jnp.float32),
            pltpu.VMEM_SHARED((_N_ACC, d), jnp.float32),
            pltpu.SemaphoreType.DMA,
            pltpu.SemaphoreType.DMA,
            pltpu.SemaphoreType.DMA,
            pltpu.SemaphoreType.DMA,
            pltpu.SemaphoreType.DMA,
        ],
    )
    def k(table_hbm, src_hbm, et_hbm, dst_hbm, out_hbm,
          gidx_v, et_v, dst_v, rows0, rows1, acc,
          sg0, sg1, ss0, ss1, so):
        c = lax.axis_index("c")
        s = lax.axis_index("s")

        def zrow(i, carry):
            for j in range(d // 16):
                rows0[i, pl.ds(j * 16, 16)] = jnp.zeros((16,), jnp.float32)
            return carry
        lax.fori_loop(0, _G, zrow, 0)
        zd = [pltpu.async_copy(rows0, acc.at[pl.ds(s * _SLAB + i * _G, _G)], so)
              for i in range(_SLAB // _G)]
        for dsc in zd:
            dsc.wait()
        plsc.subcore_barrier()

        def process(base_g, ngroups):
            pltpu.sync_copy(src_hbm.at[pl.ds(base_g, ngroups)],
                            gidx_v.at[pl.ds(0, ngroups)])
            pltpu.sync_copy(et_hbm.at[pl.ds(base_g, ngroups)],
                            et_v.at[pl.ds(0, ngroups)])
            pltpu.sync_copy(dst_hbm.at[pl.ds(base_g, ngroups)],
                            dst_v.at[pl.ds(0, ngroups)])

            def gi(g, carry):
                for j in range(_G // 16):
                    sv = gidx_v[g, pl.ds(j * 16, 16)]
                    tv = et_v[g, pl.ds(j * 16, 16)]
                    gidx_v[g, pl.ds(j * 16, 16)] = sv * _R + tv
                return carry
            lax.fori_loop(0, ngroups, gi, 0)

            pltpu.async_copy(table_hbm.at[gidx_v.at[0]], rows0, sg0)
            pltpu.async_copy(table_hbm.at[gidx_v.at[1]], rows1, sg1)

            def body(g2, carry):
                g = 2 * g2
                pltpu.make_async_copy(
                    table_hbm.at[gidx_v.at[g]], rows0, sg0).wait()
                s0 = pltpu.async_copy(rows0, acc.at[dst_v.at[g]], ss0,
                                      add=True)
                pltpu.make_async_copy(
                    table_hbm.at[gidx_v.at[g + 1]], rows1, sg1).wait()
                s1 = pltpu.async_copy(rows1, acc.at[dst_v.at[g + 1]], ss1,
                                      add=True)
                s0.wait()
                pltpu.async_copy(table_hbm.at[gidx_v.at[g + 2]], rows0, sg0)
                s1.wait()
                pltpu.async_copy(table_hbm.at[gidx_v.at[g + 3]], rows1, sg1)
                return carry
            lax.fori_loop(0, ngroups // 2 - 1, body, 0)

            g = ngroups - 2
            pltpu.make_async_copy(table_hbm.at[gidx_v.at[g]], rows0, sg0).wait()
            pltpu.sync_copy(rows0, acc.at[dst_v.at[g]], add=True)
            pltpu.make_async_copy(table_hbm.at[gidx_v.at[g + 1]], rows1, sg1).wait()
            pltpu.sync_copy(rows1, acc.at[dst_v.at[g + 1]], add=True)

        wid = s * _NC + c
        for ph in range(_GW // _PPG):
            process(wid * _GW + ph * _PPG, _PPG)

        plsc.subcore_barrier()

        od = [pltpu.async_copy(acc.at[pl.ds(s * _SLAB + i * _G, _G)],
                               out_hbm.at[c, pl.ds(s * _SLAB + i * _G, _G)], so)
              for i in range(_SLAB // _G)]
        for dsc in od:
            dsc.wait()

    return k


def _mm1_kernel(x_ref, w_ref, o_ref):
    o_ref[...] = jnp.dot(x_ref[...], w_ref[...],
                         preferred_element_type=jnp.float32)


def _layer2_kernel(inv_n, nclass, p_ref, b_ref, w_ref, o_ref):
    h = jnp.maximum((p_ref[0] + p_ref[1]) * inv_n + b_ref[...], 0.0)
    t = jnp.dot(h, w_ref[...], preferred_element_type=jnp.float32)
    z = jnp.zeros((h.shape[0], 128 - nclass), jnp.float32)
    for r in range(_R):
        o_ref[:, r * 128:r * 128 + nclass] = t[:, r * nclass:(r + 1) * nclass]
        o_ref[:, r * 128 + nclass:(r + 1) * 128] = z


def _final_kernel(inv_n, nclass, q_ref, b_ref, o_ref):
    o = (q_ref[0, :, :nclass] + q_ref[1, :, :nclass]) * inv_n + b_ref[...]
    m = jnp.max(o, axis=1, keepdims=True)
    lse = jnp.log(jnp.sum(jnp.exp(o - m), axis=1, keepdims=True)) + m
    o_ref[...] = o - lse


def kernel(x, edge_index, edge_type, W1, b1, W2, b2):
    nfeat = x.shape[1]
    nhid = W1.shape[2]
    nclass = W2.shape[2]
    inv_n = 1.0 / float(_NUM_NEIGHBORS)

    pad = _E_PAD - _E
    src = jnp.concatenate([edge_index[0].astype(jnp.int32),
                           jnp.zeros((pad,), jnp.int32)]).reshape(_E_PAD // _G, _G)
    dummy_dst = _N + (jnp.arange(pad, dtype=jnp.int32) % (_N_ACC - _N))
    dst = jnp.concatenate([edge_index[1].astype(jnp.int32),
                           dummy_dst]).reshape(_E_PAD // _G, _G)
    et = jnp.concatenate([edge_type.astype(jnp.int32),
                          jnp.zeros((pad,), jnp.int32)]).reshape(_E_PAD // _G, _G)
    W1f = W1.transpose(1, 0, 2).reshape(nfeat, _R * nhid)
    W2f = W2.transpose(1, 0, 2).reshape(nhid, _R * nclass)

    t1 = pl.pallas_call(
        _mm1_kernel,
        grid=(10,),
        in_specs=[pl.BlockSpec((_N // 10, nfeat), lambda i: (i, 0)),
                  pl.BlockSpec((nfeat, _R * nhid), lambda i: (0, 0))],
        out_specs=pl.BlockSpec((_N // 10, _R * nhid), lambda i: (i, 0)),
        out_shape=jax.ShapeDtypeStruct((_N, _R * nhid), jnp.float32),
    )(x, W1f)
    table1 = t1.reshape(_N * _R, nhid)

    p = _sc_aggregate(nhid)(table1, src, et, dst)

    t2 = pl.pallas_call(
        functools.partial(_layer2_kernel, inv_n, nclass),
        grid=(10,),
        in_specs=[pl.BlockSpec((_NC, _N_ACC // 10, nhid), lambda i: (0, i, 0)),
                  pl.BlockSpec((1, nhid), lambda i: (0, 0)),
                  pl.BlockSpec((nhid, _R * nclass), lambda i: (0, 0))],
        out_specs=pl.BlockSpec((_N_ACC // 10, _R * 128), lambda i: (i, 0)),
        out_shape=jax.ShapeDtypeStruct((_N_ACC, _R * 128), jnp.float32),
    )(p, b1.reshape(1, nhid), W2f)
    table2 = t2.reshape(_N_ACC * _R, 128)

    q = _sc_aggregate(128)(table2, src, et, dst)

    out = pl.pallas_call(
        functools.partial(_final_kernel, inv_n, nclass),
        grid=(10,),
        in_specs=[pl.BlockSpec((_NC, _N_ACC // 10, 128), lambda i: (0, i, 0)),
                  pl.BlockSpec((1, nclass), lambda i: (0, 0))],
        out_specs=pl.BlockSpec((_N_ACC // 10, nclass), lambda i: (i, 0)),
        out_shape=jax.ShapeDtypeStruct((_N_ACC, nclass), jnp.float32),
    )(q, b2.reshape(1, nclass))
    return out[:_N]


# speedup vs baseline: 2.4809x; 2.4809x over previous
import functools

import jax
import jax.numpy as jnp
from jax import lax
from jax.experimental import pallas as pl
from jax.experimental.pallas import tpu as pltpu
from jax.experimental.pallas import tpu_sc as plsc

_N = 10000
_E = 320000
_R = 4
_NUM_NEIGHBORS = 32

_NC = 2
_NS = 16
_NW = _NC * _NS

_G = 128
_E_PAD = 327680
_GW = 80
_PPG = 40
_N_ACC = 10240
_SLAB = _N_ACC // _NS


def _sc_aggregate(d: int):
    mesh = plsc.VectorSubcoreMesh(core_axis_name="c", subcore_axis_name="s")

    @functools.partial(
        pl.kernel,
        out_type=jax.ShapeDtypeStruct((_NC, _N_ACC, d), jnp.float32),
        mesh=mesh,
        scratch_types=[
            pltpu.VMEM((_PPG, _G), jnp.int32),
            pltpu.VMEM((_PPG, _G), jnp.int32),
            pltpu.VMEM((_PPG, _G), jnp.int32),
            pltpu.VMEM((_G, d), jnp.float32),
            pltpu.VMEM((_G, d), jnp.float32),
            pltpu.VMEM_SHARED((_N_ACC, d), jnp.float32),
            pltpu.SemaphoreType.DMA,
            pltpu.SemaphoreType.DMA,
            pltpu.SemaphoreType.DMA,
            pltpu.SemaphoreType.DMA,
            pltpu.SemaphoreType.DMA,
        ],
    )
    def k(table_hbm, src_hbm, et_hbm, dst_hbm, out_hbm,
          gidx_v, et_v, dst_v, rows0, rows1, acc,
          sg0, sg1, ss0, ss1, so):
        c = lax.axis_index("c")
        s = lax.axis_index("s")

        def zrow(i, carry):
            for j in range(d // 16):
                rows0[i, pl.ds(j * 16, 16)] = jnp.zeros((16,), jnp.float32)
            return carry
        lax.fori_loop(0, _G, zrow, 0)
        zd = [pltpu.async_copy(rows0, acc.at[pl.ds(s * _SLAB + i * _G, _G)], so)
              for i in range(_SLAB // _G)]
        for dsc in zd:
            dsc.wait()
        plsc.subcore_barrier()

        def process(base_g, ngroups):
            pltpu.sync_copy(src_hbm.at[pl.ds(base_g, ngroups)],
                            gidx_v.at[pl.ds(0, ngroups)])
            pltpu.sync_copy(et_hbm.at[pl.ds(base_g, ngroups)],
                            et_v.at[pl.ds(0, ngroups)])
            pltpu.sync_copy(dst_hbm.at[pl.ds(base_g, ngroups)],
                            dst_v.at[pl.ds(0, ngroups)])

            def gi(g, carry):
                for j in range(_G // 16):
                    sv = gidx_v[g, pl.ds(j * 16, 16)]
                    tv = et_v[g, pl.ds(j * 16, 16)]
                    gidx_v[g, pl.ds(j * 16, 16)] = sv * _R + tv
                return carry
            lax.fori_loop(0, ngroups, gi, 0)

            pltpu.async_copy(table_hbm.at[gidx_v.at[0]], rows0, sg0)
            pltpu.async_copy(table_hbm.at[gidx_v.at[1]], rows1, sg1)

            def body(g2, carry):
                g = 2 * g2
                pltpu.make_async_copy(
                    table_hbm.at[gidx_v.at[g]], rows0, sg0).wait()
                s0 = pltpu.async_copy(rows0, acc.at[dst_v.at[g]], ss0,
                                      add=True)
                pltpu.make_async_copy(
                    table_hbm.at[gidx_v.at[g + 1]], rows1, sg1).wait()
                s1 = pltpu.async_copy(rows1, acc.at[dst_v.at[g + 1]], ss1,
                                      add=True)
                s0.wait()
                pltpu.async_copy(table_hbm.at[gidx_v.at[g + 2]], rows0, sg0)
                s1.wait()
                pltpu.async_copy(table_hbm.at[gidx_v.at[g + 3]], rows1, sg1)
                return carry
            lax.fori_loop(0, ngroups // 2 - 1, body, 0)

            g = ngroups - 2
            pltpu.make_async_copy(table_hbm.at[gidx_v.at[g]], rows0, sg0).wait()
            pltpu.sync_copy(rows0, acc.at[dst_v.at[g]], add=True)
            pltpu.make_async_copy(table_hbm.at[gidx_v.at[g + 1]], rows1, sg1).wait()
            pltpu.sync_copy(rows1, acc.at[dst_v.at[g + 1]], add=True)

        wid = s * _NC + c
        for ph in range(_GW // _PPG):
            process(wid * _GW + ph * _PPG, _PPG)

        plsc.subcore_barrier()

        od = [pltpu.async_copy(acc.at[pl.ds(s * _SLAB + i * _G, _G)],
                               out_hbm.at[c, pl.ds(s * _SLAB + i * _G, _G)], so)
              for i in range(_SLAB // _G)]
        for dsc in od:
            dsc.wait()

    return k


def _mm1_kernel(x_ref, w_ref, o_ref):
    o_ref[...] = jnp.dot(x_ref[...], w_ref[...],
                         preferred_element_type=jnp.float32)


def _layer2_kernel(inv_n, nclass, p_ref, b_ref, w_ref, o_ref):
    h = jnp.maximum((p_ref[0] + p_ref[1]) * inv_n + b_ref[...], 0.0)
    t = jnp.dot(h, w_ref[...], preferred_element_type=jnp.float32)
    z = jnp.zeros((h.shape[0], 128 - nclass), jnp.float32)
    for r in range(_R):
        o_ref[:, r * 128:r * 128 + nclass] = t[:, r * nclass:(r + 1) * nclass]
        o_ref[:, r * 128 + nclass:(r + 1) * 128] = z


def _final_kernel(inv_n, nclass, q_ref, b_ref, o_ref):
    o = (q_ref[0, :, :nclass] + q_ref[1, :, :nclass]) * inv_n + b_ref[...]
    m = jnp.max(o, axis=1, keepdims=True)
    lse = jnp.log(jnp.sum(jnp.exp(o - m), axis=1, keepdims=True)) + m
    o_ref[...] = o - lse


def kernel(x, edge_index, edge_type, W1, b1, W2, b2):
    nfeat = x.shape[1]
    nhid = W1.shape[2]
    nclass = W2.shape[2]
    inv_n = 1.0 / float(_NUM_NEIGHBORS)

    ppw = _E // _NW
    padw = _E_PAD // _NW - ppw

    def tile_pad(arr, padvals):
        return jnp.concatenate(
            [arr.astype(jnp.int32).reshape(_NW, ppw),
             jnp.broadcast_to(padvals, (_NW, padw))],
            axis=1).reshape(_E_PAD // _G, _G)

    src = tile_pad(edge_index[0], jnp.arange(padw, dtype=jnp.int32))
    dst = tile_pad(edge_index[1], _N + jnp.arange(padw, dtype=jnp.int32))
    et = tile_pad(edge_type, jnp.zeros((padw,), jnp.int32))
    W1f = W1.transpose(1, 0, 2).reshape(nfeat, _R * nhid)
    W2f = W2.transpose(1, 0, 2).reshape(nhid, _R * nclass)

    t1 = pl.pallas_call(
        _mm1_kernel,
        grid=(10,),
        in_specs=[pl.BlockSpec((_N // 10, nfeat), lambda i: (i, 0)),
                  pl.BlockSpec((nfeat, _R * nhid), lambda i: (0, 0))],
        out_specs=pl.BlockSpec((_N // 10, _R * nhid), lambda i: (i, 0)),
        out_shape=jax.ShapeDtypeStruct((_N, _R * nhid), jnp.float32),
    )(x, W1f)
    table1 = t1.reshape(_N * _R, nhid)

    p = _sc_aggregate(nhid)(table1, src, et, dst)

    t2 = pl.pallas_call(
        functools.partial(_layer2_kernel, inv_n, nclass),
        grid=(10,),
        in_specs=[pl.BlockSpec((_NC, _N_ACC // 10, nhid), lambda i: (0, i, 0)),
                  pl.BlockSpec((1, nhid), lambda i: (0, 0)),
                  pl.BlockSpec((nhid, _R * nclass), lambda i: (0, 0))],
        out_specs=pl.BlockSpec((_N_ACC // 10, _R * 128), lambda i: (i, 0)),
        out_shape=jax.ShapeDtypeStruct((_N_ACC, _R * 128), jnp.float32),
    )(p, b1.reshape(1, nhid), W2f)
    table2 = t2.reshape(_N_ACC * _R, 128)

    q = _sc_aggregate(128)(table2, src, et, dst)

    out = pl.pallas_call(
        functools.partial(_final_kernel, inv_n, nclass),
        grid=(10,),
        in_specs=[pl.BlockSpec((_NC, _N_ACC // 10, 128), lambda i: (0, i, 0)),
                  pl.BlockSpec((1, nclass), lambda i: (0, 0))],
        out_specs=pl.BlockSpec((_N_ACC // 10, nclass), lambda i: (i, 0)),
        out_shape=jax.ShapeDtypeStruct((_N_ACC, nclass), jnp.float32),
    )(q, b2.reshape(1, nclass))
    return out[:_N]


# speedup vs baseline: 2.4921x; 1.0045x over previous
import functools

import jax
import jax.numpy as jnp
from jax import lax
from jax.experimental import pallas as pl
from jax.experimental.pallas import tpu as pltpu
from jax.experimental.pallas import tpu_sc as plsc

_N = 10000
_E = 320000
_R = 4
_NUM_NEIGHBORS = 32

_NC = 2
_NS = 16
_NW = _NC * _NS

_G = 128
_E_PAD = 327680
_GW = 80
_PPG = 40
_N_ACC = 10240
_SLAB = _N_ACC // _NS


def _sc_aggregate(d: int, stride: int):
    mesh = plsc.VectorSubcoreMesh(core_axis_name="c", subcore_axis_name="s")

    @functools.partial(
        pl.kernel,
        out_type=jax.ShapeDtypeStruct((_NC, _N_ACC, d), jnp.float32),
        mesh=mesh,
        scratch_types=[
            pltpu.VMEM((_PPG, _G), jnp.int32),
            pltpu.VMEM((_PPG, _G), jnp.int32),
            pltpu.VMEM((_PPG, _G), jnp.int32),
            pltpu.VMEM((_G, d), jnp.float32),
            pltpu.VMEM((_G, d), jnp.float32),
            pltpu.VMEM_SHARED((_N_ACC, d), jnp.float32),
            pltpu.SemaphoreType.DMA,
            pltpu.SemaphoreType.DMA,
            pltpu.SemaphoreType.DMA,
            pltpu.SemaphoreType.DMA,
            pltpu.SemaphoreType.DMA,
        ],
    )
    def k(table_hbm, src_hbm, et_hbm, dst_hbm, out_hbm,
          gidx_v, et_v, dst_v, rows0, rows1, acc,
          sg0, sg1, ss0, ss1, so):
        c = lax.axis_index("c")
        s = lax.axis_index("s")

        def zrow(i, carry):
            for j in range(d // 16):
                rows0[i, pl.ds(j * 16, 16)] = jnp.zeros((16,), jnp.float32)
            return carry
        lax.fori_loop(0, _G, zrow, 0)
        zd = [pltpu.async_copy(rows0, acc.at[pl.ds(s * _SLAB + i * _G, _G)], so)
              for i in range(_SLAB // _G)]
        for dsc in zd:
            dsc.wait()
        plsc.subcore_barrier()

        def process(base_g, ngroups):
            pltpu.sync_copy(src_hbm.at[pl.ds(base_g, ngroups)],
                            gidx_v.at[pl.ds(0, ngroups)])
            pltpu.sync_copy(et_hbm.at[pl.ds(base_g, ngroups)],
                            et_v.at[pl.ds(0, ngroups)])
            pltpu.sync_copy(dst_hbm.at[pl.ds(base_g, ngroups)],
                            dst_v.at[pl.ds(0, ngroups)])

            def gi(g, carry):
                for j in range(_G // 16):
                    sv = gidx_v[g, pl.ds(j * 16, 16)]
                    tv = et_v[g, pl.ds(j * 16, 16)]
                    gidx_v[g, pl.ds(j * 16, 16)] = tv * stride + sv
                return carry
            lax.fori_loop(0, ngroups, gi, 0)

            pltpu.async_copy(table_hbm.at[gidx_v.at[0]], rows0, sg0)
            pltpu.async_copy(table_hbm.at[gidx_v.at[1]], rows1, sg1)

            def body(g2, carry):
                g = 2 * g2
                pltpu.make_async_copy(
                    table_hbm.at[gidx_v.at[g]], rows0, sg0).wait()
                s0 = pltpu.async_copy(rows0, acc.at[dst_v.at[g]], ss0,
                                      add=True)
                pltpu.make_async_copy(
                    table_hbm.at[gidx_v.at[g + 1]], rows1, sg1).wait()
                s1 = pltpu.async_copy(rows1, acc.at[dst_v.at[g + 1]], ss1,
                                      add=True)
                s0.wait()
                pltpu.async_copy(table_hbm.at[gidx_v.at[g + 2]], rows0, sg0)
                s1.wait()
                pltpu.async_copy(table_hbm.at[gidx_v.at[g + 3]], rows1, sg1)
                return carry
            lax.fori_loop(0, ngroups // 2 - 1, body, 0)

            g = ngroups - 2
            pltpu.make_async_copy(table_hbm.at[gidx_v.at[g]], rows0, sg0).wait()
            pltpu.sync_copy(rows0, acc.at[dst_v.at[g]], add=True)
            pltpu.make_async_copy(table_hbm.at[gidx_v.at[g + 1]], rows1, sg1).wait()
            pltpu.sync_copy(rows1, acc.at[dst_v.at[g + 1]], add=True)

        wid = s * _NC + c
        for ph in range(_GW // _PPG):
            process(wid * _GW + ph * _PPG, _PPG)

        plsc.subcore_barrier()

        od = [pltpu.async_copy(acc.at[pl.ds(s * _SLAB + i * _G, _G)],
                               out_hbm.at[c, pl.ds(s * _SLAB + i * _G, _G)], so)
              for i in range(_SLAB // _G)]
        for dsc in od:
            dsc.wait()

    return k


def _mm1_kernel(x_ref, w_ref, o_ref):
    o_ref[...] = jnp.dot(x_ref[...], w_ref[0],
                         preferred_element_type=jnp.float32)


def _layer2_kernel(inv_n, nclass, p_ref, b_ref, w_ref, o_ref):
    h = jnp.maximum((p_ref[0] + p_ref[1]) * inv_n + b_ref[...], 0.0)
    o_ref[:, :nclass] = jnp.dot(h, w_ref[0],
                                preferred_element_type=jnp.float32)
    o_ref[:, nclass:] = jnp.zeros((h.shape[0], 128 - nclass), jnp.float32)


def _final_kernel(inv_n, nclass, q_ref, b_ref, o_ref):
    o = (q_ref[0, :, :nclass] + q_ref[1, :, :nclass]) * inv_n + b_ref[...]
    m = jnp.max(o, axis=1, keepdims=True)
    lse = jnp.log(jnp.sum(jnp.exp(o - m), axis=1, keepdims=True)) + m
    o_ref[...] = o - lse


def kernel(x, edge_index, edge_type, W1, b1, W2, b2):
    nfeat = x.shape[1]
    nhid = W1.shape[2]
    nclass = W2.shape[2]
    inv_n = 1.0 / float(_NUM_NEIGHBORS)

    ppw = _E // _NW
    padw = _E_PAD // _NW - ppw

    def tile_pad(arr, padvals):
        return jnp.concatenate(
            [arr.astype(jnp.int32).reshape(_NW, ppw),
             jnp.broadcast_to(padvals, (_NW, padw))],
            axis=1).reshape(_E_PAD // _G, _G)

    src = tile_pad(edge_index[0], jnp.arange(padw, dtype=jnp.int32))
    dst = tile_pad(edge_index[1], _N + jnp.arange(padw, dtype=jnp.int32))
    et = tile_pad(edge_type, jnp.zeros((padw,), jnp.int32))

    blk = _N // 10
    table1 = pl.pallas_call(
        _mm1_kernel,
        grid=(_R, 10),
        in_specs=[pl.BlockSpec((blk, nfeat), lambda r, i: (i, 0)),
                  pl.BlockSpec((1, nfeat, nhid), lambda r, i: (r, 0, 0))],
        out_specs=pl.BlockSpec((blk, nhid), lambda r, i: (r * 10 + i, 0)),
        out_shape=jax.ShapeDtypeStruct((_R * _N, nhid), jnp.float32),
    )(x, W1)

    p = _sc_aggregate(nhid, _N)(table1, src, et, dst)

    blk2 = _N_ACC // 10
    table2 = pl.pallas_call(
        functools.partial(_layer2_kernel, inv_n, nclass),
        grid=(_R, 10),
        in_specs=[pl.BlockSpec((_NC, blk2, nhid), lambda r, i: (0, i, 0)),
                  pl.BlockSpec((1, nhid), lambda r, i: (0, 0)),
                  pl.BlockSpec((1, nhid, nclass), lambda r, i: (r, 0, 0))],
        out_specs=pl.BlockSpec((blk2, 128), lambda r, i: (r * 10 + i, 0)),
        out_shape=jax.ShapeDtypeStruct((_R * _N_ACC, 128), jnp.float32),
    )(p, b1.reshape(1, nhid), W2)

    q = _sc_aggregate(128, _N_ACC)(table2, src, et, dst)

    return pl.pallas_call(
        functools.partial(_final_kernel, inv_n, nclass),
        grid=(10,),
        in_specs=[pl.BlockSpec((_NC, blk, 128), lambda i: (0, i, 0)),
                  pl.BlockSpec((1, nclass), lambda i: (0, 0))],
        out_specs=pl.BlockSpec((blk, nclass), lambda i: (i, 0)),
        out_shape=jax.ShapeDtypeStruct((_N, nclass), jnp.float32),
    )(q, b2.reshape(1, nclass))


# speedup vs baseline: 2.5551x; 1.0253x over previous
import functools

import jax
import jax.numpy as jnp
from jax import lax
from jax.experimental import pallas as pl
from jax.experimental.pallas import tpu as pltpu
from jax.experimental.pallas import tpu_sc as plsc

_N = 10000
_E = 320000
_R = 4
_NUM_NEIGHBORS = 32

_NC = 2
_NS = 16
_NW = _NC * _NS

_G = 128
_E_PAD = 327680
_GW = 80
_PPG = 40
_N_ACC = 10240
_SLAB = _N_ACC // _NS


def _sc_aggregate(d: int, stride: int):
    mesh = plsc.VectorSubcoreMesh(core_axis_name="c", subcore_axis_name="s")

    @functools.partial(
        pl.kernel,
        out_type=jax.ShapeDtypeStruct((_NC, _N_ACC, d), jnp.float32),
        mesh=mesh,
        scratch_types=[
            pltpu.VMEM((_PPG, _G), jnp.int32),
            pltpu.VMEM((_PPG, _G), jnp.int32),
            pltpu.VMEM((_PPG, _G), jnp.int32),
            pltpu.VMEM((_G, d), jnp.float32),
            pltpu.VMEM((_G, d), jnp.float32),
            pltpu.VMEM_SHARED((_N_ACC, d), jnp.float32),
            pltpu.SemaphoreType.DMA,
            pltpu.SemaphoreType.DMA,
            pltpu.SemaphoreType.DMA,
            pltpu.SemaphoreType.DMA,
            pltpu.SemaphoreType.DMA,
        ],
    )
    def k(table_hbm, src_hbm, et_hbm, dst_hbm, out_hbm,
          gidx_v, et_v, dst_v, rows0, rows1, acc,
          sg0, sg1, ss0, ss1, so):
        c = lax.axis_index("c")
        s = lax.axis_index("s")

        def zrow(i, carry):
            for j in range(d // 16):
                rows0[i, pl.ds(j * 16, 16)] = jnp.zeros((16,), jnp.float32)
            return carry
        lax.fori_loop(0, _G, zrow, 0)
        zd = [pltpu.async_copy(rows0, acc.at[pl.ds(s * _SLAB + i * _G, _G)], so)
              for i in range(_SLAB // _G)]
        for dsc in zd:
            dsc.wait()
        plsc.subcore_barrier()

        def process(base_g, ngroups):
            pltpu.sync_copy(src_hbm.at[pl.ds(base_g, ngroups)],
                            gidx_v.at[pl.ds(0, ngroups)])
            pltpu.sync_copy(et_hbm.at[pl.ds(base_g, ngroups)],
                            et_v.at[pl.ds(0, ngroups)])
            pltpu.sync_copy(dst_hbm.at[pl.ds(base_g, ngroups)],
                            dst_v.at[pl.ds(0, ngroups)])

            def gi(g, carry):
                for j in range(_G // 16):
                    sv = gidx_v[g, pl.ds(j * 16, 16)]
                    tv = et_v[g, pl.ds(j * 16, 16)]
                    gidx_v[g, pl.ds(j * 16, 16)] = tv * stride + sv
                return carry
            lax.fori_loop(0, ngroups, gi, 0)

            pltpu.async_copy(table_hbm.at[gidx_v.at[0]], rows0, sg0)
            pltpu.async_copy(table_hbm.at[gidx_v.at[1]], rows1, sg1)

            def body(g2, carry):
                g = 2 * g2
                pltpu.make_async_copy(
                    table_hbm.at[gidx_v.at[g]], rows0, sg0).wait()
                s0 = pltpu.async_copy(rows0, acc.at[dst_v.at[g]], ss0,
                                      add=True)
                pltpu.make_async_copy(
                    table_hbm.at[gidx_v.at[g + 1]], rows1, sg1).wait()
                s1 = pltpu.async_copy(rows1, acc.at[dst_v.at[g + 1]], ss1,
                                      add=True)
                s0.wait()
                pltpu.async_copy(table_hbm.at[gidx_v.at[g + 2]], rows0, sg0)
                s1.wait()
                pltpu.async_copy(table_hbm.at[gidx_v.at[g + 3]], rows1, sg1)
                return carry
            lax.fori_loop(0, ngroups // 2 - 1, body, 0)

            g = ngroups - 2
            pltpu.make_async_copy(table_hbm.at[gidx_v.at[g]], rows0, sg0).wait()
            pltpu.sync_copy(rows0, acc.at[dst_v.at[g]], add=True)
            pltpu.make_async_copy(table_hbm.at[gidx_v.at[g + 1]], rows1, sg1).wait()
            pltpu.sync_copy(rows1, acc.at[dst_v.at[g + 1]], add=True)

        wid = s * _NC + c
        for ph in range(_GW // _PPG):
            process(wid * _GW + ph * _PPG, _PPG)

        plsc.subcore_barrier()

        od = [pltpu.async_copy(acc.at[pl.ds(s * _SLAB + i * _G, _G)],
                               out_hbm.at[c, pl.ds(s * _SLAB + i * _G, _G)], so)
              for i in range(_SLAB // _G)]
        for dsc in od:
            dsc.wait()

    return k


def _mm1_kernel(x_ref, w_ref, o_ref):
    o_ref[...] = jnp.dot(x_ref[...], w_ref[0],
                         preferred_element_type=jnp.float32)


def _layer2_kernel(inv_n, nclass, p_ref, b_ref, w_ref, o_ref):
    h = jnp.maximum((p_ref[0] + p_ref[1]) * inv_n + b_ref[...], 0.0)
    o_ref[:, :nclass] = jnp.dot(h, w_ref[0],
                                preferred_element_type=jnp.float32)
    o_ref[:, nclass:] = jnp.zeros((h.shape[0], 128 - nclass), jnp.float32)


def _final_kernel(inv_n, nclass, q_ref, b_ref, o_ref):
    o = (q_ref[0, :, :nclass] + q_ref[1, :, :nclass]) * inv_n + b_ref[...]
    m = jnp.max(o, axis=1, keepdims=True)
    lse = jnp.log(jnp.sum(jnp.exp(o - m), axis=1, keepdims=True)) + m
    o_ref[...] = o - lse


def kernel(x, edge_index, edge_type, W1, b1, W2, b2):
    nfeat = x.shape[1]
    nhid = W1.shape[2]
    nclass = W2.shape[2]
    inv_n = 1.0 / float(_NUM_NEIGHBORS)

    ppw = _E // _NW
    padw = _E_PAD // _NW - ppw

    def tile_pad(arr, padvals):
        return jnp.concatenate(
            [arr.astype(jnp.int32).reshape(_NW, ppw),
             jnp.broadcast_to(padvals, (_NW, padw))],
            axis=1).reshape(_E_PAD // _G, _G)

    src = tile_pad(edge_index[0], jnp.arange(padw, dtype=jnp.int32))
    dst = tile_pad(edge_index[1], _N + jnp.arange(padw, dtype=jnp.int32))
    et = tile_pad(edge_type, jnp.zeros((padw,), jnp.int32))

    blk = _N // 10
    table1 = pl.pallas_call(
        _mm1_kernel,
        grid=(10, _R),
        in_specs=[pl.BlockSpec((blk, nfeat), lambda i, r: (i, 0)),
                  pl.BlockSpec((1, nfeat, nhid), lambda i, r: (r, 0, 0))],
        out_specs=pl.BlockSpec((blk, nhid), lambda i, r: (r * 10 + i, 0)),
        out_shape=jax.ShapeDtypeStruct((_R * _N, nhid), jnp.float32),
    )(x, W1)

    p = _sc_aggregate(nhid, _N)(table1, src, et, dst)

    blk2 = _N_ACC // 10
    table2 = pl.pallas_call(
        functools.partial(_layer2_kernel, inv_n, nclass),
        grid=(10, _R),
        in_specs=[pl.BlockSpec((_NC, blk2, nhid), lambda i, r: (0, i, 0)),
                  pl.BlockSpec((1, nhid), lambda i, r: (0, 0)),
                  pl.BlockSpec((1, nhid, nclass), lambda i, r: (r, 0, 0))],
        out_specs=pl.BlockSpec((blk2, 128), lambda i, r: (r * 10 + i, 0)),
        out_shape=jax.ShapeDtypeStruct((_R * _N_ACC, 128), jnp.float32),
    )(p, b1.reshape(1, nhid), W2)

    q = _sc_aggregate(128, _N_ACC)(table2, src, et, dst)

    return pl.pallas_call(
        functools.partial(_final_kernel, inv_n, nclass),
        grid=(10,),
        in_specs=[pl.BlockSpec((_NC, blk, 128), lambda i: (0, i, 0)),
                  pl.BlockSpec((1, nclass), lambda i: (0, 0))],
        out_specs=pl.BlockSpec((blk, nclass), lambda i: (i, 0)),
        out_shape=jax.ShapeDtypeStruct((_N, nclass), jnp.float32),
    )(q, b2.reshape(1, nclass))


# speedup vs baseline: 2.5874x; 1.0127x over previous
import functools

import jax
import jax.numpy as jnp
from jax import lax
from jax.experimental import pallas as pl
from jax.experimental.pallas import tpu as pltpu
from jax.experimental.pallas import tpu_sc as plsc

_N = 10000
_E = 320000
_R = 4
_NUM_NEIGHBORS = 32

_NC = 2
_NS = 16
_NW = _NC * _NS

_G = 128
_E_PAD = 327680
_GW = 80
_PPG = 40
_N_ACC = 10240
_SLAB = _N_ACC // _NS


def _sc_aggregate(d: int, stride: int):
    mesh = plsc.VectorSubcoreMesh(core_axis_name="c", subcore_axis_name="s")

    @functools.partial(
        pl.kernel,
        out_type=jax.ShapeDtypeStruct((_NC, _N_ACC, d), jnp.float32),
        mesh=mesh,
        scratch_types=[
            pltpu.VMEM((_PPG, _G), jnp.int32),
            pltpu.VMEM((_PPG, _G), jnp.int32),
            pltpu.VMEM((_PPG, _G), jnp.int32),
            pltpu.VMEM((_G, d), jnp.float32),
            pltpu.VMEM((_G, d), jnp.float32),
            pltpu.VMEM_SHARED((_N_ACC, d), jnp.float32),
            pltpu.SemaphoreType.DMA,
            pltpu.SemaphoreType.DMA,
            pltpu.SemaphoreType.DMA,
            pltpu.SemaphoreType.DMA,
            pltpu.SemaphoreType.DMA,
            pltpu.SemaphoreType.DMA,
            pltpu.SemaphoreType.DMA,
        ],
    )
    def k(table_hbm, src_hbm, et_hbm, dst_hbm, out_hbm,
          gidx_v, et_v, dst_v, rows0, rows1, acc,
          sg0a, sg0b, sg1a, sg1b, ss0, ss1, so):
        c = lax.axis_index("c")
        s = lax.axis_index("s")

        def zrow(i, carry):
            for j in range(d // 16):
                rows0[i, pl.ds(j * 16, 16)] = jnp.zeros((16,), jnp.float32)
            return carry
        lax.fori_loop(0, _G, zrow, 0)
        zd = [pltpu.async_copy(rows0, acc.at[pl.ds(s * _SLAB + i * _G, _G)], so)
              for i in range(_SLAB // _G)]
        for dsc in zd:
            dsc.wait()
        plsc.subcore_barrier()

        def process(base_g, ngroups):
            pltpu.sync_copy(src_hbm.at[pl.ds(base_g, ngroups)],
                            gidx_v.at[pl.ds(0, ngroups)])
            pltpu.sync_copy(et_hbm.at[pl.ds(base_g, ngroups)],
                            et_v.at[pl.ds(0, ngroups)])
            pltpu.sync_copy(dst_hbm.at[pl.ds(base_g, ngroups)],
                            dst_v.at[pl.ds(0, ngroups)])

            def gi(g, carry):
                for j in range(_G // 16):
                    sv = gidx_v[g, pl.ds(j * 16, 16)]
                    tv = et_v[g, pl.ds(j * 16, 16)]
                    gidx_v[g, pl.ds(j * 16, 16)] = tv * stride + sv
                return carry
            lax.fori_loop(0, ngroups, gi, 0)

            h = _G // 2

            def gather(g, rv, sa, sb):
                pltpu.async_copy(table_hbm.at[gidx_v.at[g, pl.ds(0, h)]],
                                 rv.at[pl.ds(0, h)], sa)
                pltpu.async_copy(table_hbm.at[gidx_v.at[g, pl.ds(h, h)]],
                                 rv.at[pl.ds(h, h)], sb)

            def gather_wait(g, rv, sa, sb):
                pltpu.make_async_copy(table_hbm.at[gidx_v.at[g, pl.ds(0, h)]],
                                      rv.at[pl.ds(0, h)], sa).wait()
                pltpu.make_async_copy(table_hbm.at[gidx_v.at[g, pl.ds(h, h)]],
                                      rv.at[pl.ds(h, h)], sb).wait()

            gather(0, rows0, sg0a, sg0b)
            gather(1, rows1, sg1a, sg1b)

            def body(g2, carry):
                g = 2 * g2
                gather_wait(g, rows0, sg0a, sg0b)
                s0 = pltpu.async_copy(rows0, acc.at[dst_v.at[g]], ss0,
                                      add=True)
                gather_wait(g + 1, rows1, sg1a, sg1b)
                s1 = pltpu.async_copy(rows1, acc.at[dst_v.at[g + 1]], ss1,
                                      add=True)
                s0.wait()
                gather(g + 2, rows0, sg0a, sg0b)
                s1.wait()
                gather(g + 3, rows1, sg1a, sg1b)
                return carry
            lax.fori_loop(0, ngroups // 2 - 1, body, 0)

            g = ngroups - 2
            gather_wait(g, rows0, sg0a, sg0b)
            pltpu.sync_copy(rows0, acc.at[dst_v.at[g]], add=True)
            gather_wait(g + 1, rows1, sg1a, sg1b)
            pltpu.sync_copy(rows1, acc.at[dst_v.at[g + 1]], add=True)

        wid = s * _NC + c
        for ph in range(_GW // _PPG):
            process(wid * _GW + ph * _PPG, _PPG)

        plsc.subcore_barrier()

        od = [pltpu.async_copy(acc.at[pl.ds(s * _SLAB + i * _G, _G)],
                               out_hbm.at[c, pl.ds(s * _SLAB + i * _G, _G)], so)
              for i in range(_SLAB // _G)]
        for dsc in od:
            dsc.wait()

    return k


def _mm1_kernel(x_ref, w_ref, o_ref):
    o_ref[...] = jnp.dot(x_ref[...], w_ref[0],
                         preferred_element_type=jnp.float32)


def _layer2_kernel(inv_n, nclass, p_ref, b_ref, w_ref, o_ref):
    h = jnp.maximum((p_ref[0] + p_ref[1]) * inv_n + b_ref[...], 0.0)
    o_ref[:, :nclass] = jnp.dot(h, w_ref[0],
                                preferred_element_type=jnp.float32)
    o_ref[:, nclass:] = jnp.zeros((h.shape[0], 128 - nclass), jnp.float32)


def _final_kernel(inv_n, nclass, q_ref, b_ref, o_ref):
    o = (q_ref[0, :, :nclass] + q_ref[1, :, :nclass]) * inv_n + b_ref[...]
    m = jnp.max(o, axis=1, keepdims=True)
    lse = jnp.log(jnp.sum(jnp.exp(o - m), axis=1, keepdims=True)) + m
    o_ref[...] = o - lse


def kernel(x, edge_index, edge_type, W1, b1, W2, b2):
    nfeat = x.shape[1]
    nhid = W1.shape[2]
    nclass = W2.shape[2]
    inv_n = 1.0 / float(_NUM_NEIGHBORS)

    ppw = _E // _NW
    padw = _E_PAD // _NW - ppw

    def tile_pad(arr, padvals):
        return jnp.concatenate(
            [arr.astype(jnp.int32).reshape(_NW, ppw),
             jnp.broadcast_to(padvals, (_NW, padw))],
            axis=1).reshape(_E_PAD // _G, _G)

    src = tile_pad(edge_index[0], jnp.arange(padw, dtype=jnp.int32))
    dst = tile_pad(edge_index[1], _N + jnp.arange(padw, dtype=jnp.int32))
    et = tile_pad(edge_type, jnp.zeros((padw,), jnp.int32))

    blk = _N // 10
    table1 = pl.pallas_call(
        _mm1_kernel,
        grid=(10, _R),
        in_specs=[pl.BlockSpec((blk, nfeat), lambda i, r: (i, 0)),
                  pl.BlockSpec((1, nfeat, nhid), lambda i, r: (r, 0, 0))],
        out_specs=pl.BlockSpec((blk, nhid), lambda i, r: (r * 10 + i, 0)),
        out_shape=jax.ShapeDtypeStruct((_R * _N, nhid), jnp.float32),
    )(x, W1)

    p = _sc_aggregate(nhid, _N)(table1, src, et, dst)

    blk2 = _N_ACC // 10
    table2 = pl.pallas_call(
        functools.partial(_layer2_kernel, inv_n, nclass),
        grid=(10, _R),
        in_specs=[pl.BlockSpec((_NC, blk2, nhid), lambda i, r: (0, i, 0)),
                  pl.BlockSpec((1, nhid), lambda i, r: (0, 0)),
                  pl.BlockSpec((1, nhid, nclass), lambda i, r: (r, 0, 0))],
        out_specs=pl.BlockSpec((blk2, 128), lambda i, r: (r * 10 + i, 0)),
        out_shape=jax.ShapeDtypeStruct((_R * _N_ACC, 128), jnp.float32),
    )(p, b1.reshape(1, nhid), W2)

    q = _sc_aggregate(128, _N_ACC)(table2, src, et, dst)

    return pl.pallas_call(
        functools.partial(_final_kernel, inv_n, nclass),
        grid=(10,),
        in_specs=[pl.BlockSpec((_NC, blk, 128), lambda i: (0, i, 0)),
                  pl.BlockSpec((1, nclass), lambda i: (0, 0))],
        out_specs=pl.BlockSpec((blk, nclass), lambda i: (i, 0)),
        out_shape=jax.ShapeDtypeStruct((_N, nclass), jnp.float32),
    )(q, b2.reshape(1, nclass))
